# baseline (speedup 1.0000x reference)
import functools

import jax
import jax.numpy as jnp
from jax import lax
from jax.experimental import pallas as pl
from jax.experimental.pallas import tpu as pltpu
from jax.experimental.pallas import tpu_sc as plsc

B, S, V, H = 1024, 200, 1000000, 128

NC, NS = 2, 16
NW = NC * NS
NUM_IDS = B * S
IDS_PER_W = NUM_IDS // NW
CHUNK = 80
NCHUNK = IDS_PER_W // CHUNK
NBUF = 8
DEPTH = NBUF // 2


def _gather_body(ids_hbm, table_hbm, out_hbm, ids_v, *rest):
    bufs, sem_g, sem_w = rest[:NBUF], rest[NBUF:2 * NBUF], rest[2 * NBUF:]
    wid = lax.axis_index("s") * NC + lax.axis_index("c")
    out_base = wid * IDS_PER_W
    pltpu.sync_copy(ids_hbm.at[wid], ids_v)

    def out_slice(j):
        return out_hbm.at[pl.ds(out_base + j * CHUNK, CHUNK)]

    for b in range(DEPTH):
        pltpu.async_copy(table_hbm.at[ids_v.at[b]], bufs[b], sem_g[b])

    @pl.loop(0, NCHUNK, step=NBUF)
    def _(j0):
        for b in range(NBUF):
            j = j0 + b
            pltpu.make_async_copy(table_hbm.at[ids_v.at[j]],
                                  bufs[b], sem_g[b]).wait()
            pltpu.async_copy(bufs[b], out_slice(j), sem_w[b])
            jn = j + DEPTH
            bn = (b + DEPTH) % NBUF

            @pl.when(jn < NCHUNK)
            def _():
                @pl.when(jn >= NBUF)
                def _():
                    pltpu.make_async_copy(bufs[bn], out_slice(jn),
                                          sem_w[bn]).wait()
                pltpu.async_copy(table_hbm.at[ids_v.at[jn]], bufs[bn],
                                 sem_g[bn])

    for b in range(NBUF):
        j = NCHUNK - NBUF + b
        pltpu.make_async_copy(bufs[b], out_slice(j), sem_w[b]).wait()


@functools.cache
def _sc_gather():
    return pl.kernel(
        _gather_body,
        out_type=jax.ShapeDtypeStruct((NUM_IDS, H), jnp.float32),
        mesh=plsc.VectorSubcoreMesh(core_axis_name="c", subcore_axis_name="s",
                                    num_cores=NC, num_subcores=NS),
        scratch_types=(
            [pltpu.VMEM((NCHUNK, CHUNK), jnp.int32)]
            + [pltpu.VMEM((CHUNK, H), jnp.float32) for _ in range(NBUF)]
            + [pltpu.SemaphoreType.DMA for _ in range(2 * NBUF)]
        ),
    )


RB = 16
NBLK = B // RB


def _mlp_body(x_ref, v_ref, pec_ref, w1_ref, w1bc_ref, b1c_ref,
              g1_ref, bb1_ref, w2_ref, g2_ref, bb2_ref, jm_ref, o_ref):
    jm = jm_ref[...]
    x = x_ref[...].reshape(RB * S, H)
    h = jnp.dot(x, w1_ref[...], preferred_element_type=jnp.float32)
    vb = v_ref[...][:, :, None] * w1bc_ref[...].reshape(1, 1, H)
    h = h + (vb.reshape(RB * S, H) + b1c_ref[...].reshape(1, H))
    hc = h - jnp.dot(h, jm, preferred_element_type=jnp.float32)
    var = jnp.dot(hc * hc, jm, preferred_element_type=jnp.float32)
    h = hc * lax.rsqrt(var + 1e-5)
    h = h * g1_ref[...].reshape(1, H) + bb1_ref[...].reshape(1, H)
    h = h * jax.nn.sigmoid(1.702 * h)
    h = jnp.dot(h, w2_ref[...], preferred_element_type=jnp.float32)
    emb = (h.reshape(RB, S, H) + pec_ref[...][None]).reshape(RB * S, H)
    ec = emb - jnp.dot(emb, jm, preferred_element_type=jnp.float32)
    var2 = jnp.dot(ec * ec, jm, preferred_element_type=jnp.float32)
    o_ref[...] = (ec * lax.rsqrt(var2 + 1e-12) * g2_ref[...].reshape(1, H)
                  + bb2_ref[...].reshape(1, H)).reshape(RB, S, H)


def _const_spec(shape):
    return pl.BlockSpec(shape, lambda i: tuple(0 for _ in shape))


_tc_mlp = pl.pallas_call(
    _mlp_body,
    grid=(NBLK,),
    in_specs=[
        pl.BlockSpec((RB, S, H), lambda i: (i, 0, 0)),
        pl.BlockSpec((RB, S), lambda i: (i, 0)),
        _const_spec((S, H)),
        _const_spec((H, H)),
        _const_spec((H,)),
        _const_spec((H,)),
        _const_spec((H,)),
        _const_spec((H,)),
        _const_spec((H, H)),
        _const_spec((H,)),
        _const_spec((H,)),
        _const_spec((H, H)),
    ],
    out_specs=pl.BlockSpec((RB, S, H), lambda i: (i, 0, 0)),
    out_shape=jax.ShapeDtypeStruct((B, S, H), jnp.float32),
    compiler_params=pltpu.CompilerParams(
        dimension_semantics=("arbitrary",),
    ),
)


def kernel(input_ids, values, word_emb, cat_fc_w, cat_fc_b, cat_ln_g, cat_ln_b,
           cat_proj_w, cat_proj_b, pos_emb, tok_emb, ln_g, ln_b):
    ids = input_ids.astype(jnp.int32).reshape(NW, NCHUNK, CHUNK)
    vals = values.astype(jnp.float32)
    jm = jnp.full((H, H), 1.0 / H, jnp.float32)
    pe_eff = pos_emb[:S] + tok_emb[0] + cat_proj_b
    pec = pe_eff - pe_eff.mean(-1, keepdims=True)
    w1a, w1b = cat_fc_w[:H], cat_fc_w[H]
    w1bc = w1b - w1b.mean()
    b1c = cat_fc_b - cat_fc_b.mean()
    gathered = _sc_gather()(ids, word_emb)
    return _tc_mlp(
        gathered.reshape(B, S, H), vals, pec,
        w1a, w1bc, b1c, cat_ln_g, cat_ln_b,
        cat_proj_w, ln_g, ln_b, jm,
    )


# speedup vs baseline: 1.0828x; 1.0828x over previous
import functools

import jax
import jax.numpy as jnp
from jax import lax
from jax.experimental import pallas as pl
from jax.experimental.pallas import tpu as pltpu
from jax.experimental.pallas import tpu_sc as plsc

B, S, V, H = 1024, 200, 1000000, 128

NC, NS = 2, 16
NW = NC * NS
NUM_IDS = B * S
IDS_PER_W = NUM_IDS // NW
CHUNK = 80
NCHUNK = IDS_PER_W // CHUNK
NBUF = 8
DEPTH = NBUF // 2


def _gather_body(ids_hbm, table_hbm, out_hbm, ids_v, *rest):
    bufs, sem_g, sem_w = rest[:NBUF], rest[NBUF:2 * NBUF], rest[2 * NBUF:]
    wid = lax.axis_index("s") * NC + lax.axis_index("c")
    out_base = wid * IDS_PER_W
    pltpu.sync_copy(ids_hbm.at[wid], ids_v)

    def out_slice(j):
        return out_hbm.at[pl.ds(out_base + j * CHUNK, CHUNK)]

    for b in range(DEPTH):
        pltpu.async_copy(table_hbm.at[ids_v.at[b]], bufs[b], sem_g[b])

    @pl.loop(0, NCHUNK, step=NBUF)
    def _(j0):
        for b in range(NBUF):
            j = j0 + b
            pltpu.make_async_copy(table_hbm.at[ids_v.at[j]],
                                  bufs[b], sem_g[b]).wait()
            pltpu.async_copy(bufs[b], out_slice(j), sem_w[b])
            jn = j + DEPTH
            bn = (b + DEPTH) % NBUF

            @pl.when(jn < NCHUNK)
            def _():
                @pl.when(jn >= NBUF)
                def _():
                    pltpu.make_async_copy(bufs[bn], out_slice(jn),
                                          sem_w[bn]).wait()
                pltpu.async_copy(table_hbm.at[ids_v.at[jn]], bufs[bn],
                                 sem_g[bn])

    for b in range(NBUF):
        j = NCHUNK - NBUF + b
        pltpu.make_async_copy(bufs[b], out_slice(j), sem_w[b]).wait()


@functools.cache
def _sc_gather():
    return pl.kernel(
        _gather_body,
        out_type=jax.ShapeDtypeStruct((NUM_IDS, H), jnp.float32),
        mesh=plsc.VectorSubcoreMesh(core_axis_name="c", subcore_axis_name="s",
                                    num_cores=NC, num_subcores=NS),
        scratch_types=(
            [pltpu.VMEM((NCHUNK, CHUNK), jnp.int32)]
            + [pltpu.VMEM((CHUNK, H), jnp.float32) for _ in range(NBUF)]
            + [pltpu.SemaphoreType.DMA for _ in range(2 * NBUF)]
        ),
    )


RB = 64
NBLK = B // RB


def _mlp_body(x_ref, v_ref, pec_ref, w1_ref, w1bc_ref, b1c_ref,
              g1_ref, bb1_ref, w2_ref, g2_ref, bb2_ref, jm_ref, o_ref):
    jm = jm_ref[...]
    x = x_ref[...].reshape(RB * S, H)
    h = jnp.dot(x, w1_ref[...], preferred_element_type=jnp.float32)
    vb = v_ref[...][:, :, None] * w1bc_ref[...].reshape(1, 1, H)
    h = h + (vb.reshape(RB * S, H) + b1c_ref[...].reshape(1, H))
    hc = h - jnp.dot(h, jm, preferred_element_type=jnp.float32)
    var = jnp.dot(hc * hc, jm, preferred_element_type=jnp.float32)
    h = hc * lax.rsqrt(var + 1e-5)
    h = h * g1_ref[...].reshape(1, H) + bb1_ref[...].reshape(1, H)
    h = h * jax.nn.sigmoid(1.702 * h)
    h = jnp.dot(h, w2_ref[...], preferred_element_type=jnp.float32)
    emb = (h.reshape(RB, S, H) + pec_ref[...][None]).reshape(RB * S, H)
    ec = emb - jnp.dot(emb, jm, preferred_element_type=jnp.float32)
    var2 = jnp.dot(ec * ec, jm, preferred_element_type=jnp.float32)
    o_ref[...] = (ec * lax.rsqrt(var2 + 1e-12) * g2_ref[...].reshape(1, H)
                  + bb2_ref[...].reshape(1, H)).reshape(RB, S, H)


def _const_spec(shape):
    return pl.BlockSpec(shape, lambda i: tuple(0 for _ in shape))


_tc_mlp = pl.pallas_call(
    _mlp_body,
    grid=(NBLK,),
    in_specs=[
        pl.BlockSpec((RB, S, H), lambda i: (i, 0, 0)),
        pl.BlockSpec((RB, S), lambda i: (i, 0)),
        _const_spec((S, H)),
        _const_spec((H, H)),
        _const_spec((H,)),
        _const_spec((H,)),
        _const_spec((H,)),
        _const_spec((H,)),
        _const_spec((H, H)),
        _const_spec((H,)),
        _const_spec((H,)),
        _const_spec((H, H)),
    ],
    out_specs=pl.BlockSpec((RB, S, H), lambda i: (i, 0, 0)),
    out_shape=jax.ShapeDtypeStruct((B, S, H), jnp.float32),
    compiler_params=pltpu.CompilerParams(
        dimension_semantics=("arbitrary",),
    ),
)


def kernel(input_ids, values, word_emb, cat_fc_w, cat_fc_b, cat_ln_g, cat_ln_b,
           cat_proj_w, cat_proj_b, pos_emb, tok_emb, ln_g, ln_b):
    ids = input_ids.astype(jnp.int32).reshape(NW, NCHUNK, CHUNK)
    vals = values.astype(jnp.float32)
    jm = jnp.full((H, H), 1.0 / H, jnp.float32)
    pe_eff = pos_emb[:S] + tok_emb[0] + cat_proj_b
    pec = pe_eff - pe_eff.mean(-1, keepdims=True)
    w1a, w1b = cat_fc_w[:H], cat_fc_w[H]
    w1bc = w1b - w1b.mean()
    b1c = cat_fc_b - cat_fc_b.mean()
    gathered = _sc_gather()(ids, word_emb)
    return _tc_mlp(
        gathered.reshape(B, S, H), vals, pec,
        w1a, w1bc, b1c, cat_ln_g, cat_ln_b,
        cat_proj_w, ln_g, ln_b, jm,
    )
